# baseline (speedup 1.0000x reference)
import functools

import jax
import jax.numpy as jnp
from jax import lax
from jax.experimental import pallas as pl
from jax.experimental.pallas import tpu as pltpu
from jax.experimental.pallas import tpu_sc as plsc

_DIM = 64
_NE = 8192
_W = 2048
_BM = 768


def _dist_argmin_body(x_ref, x2_ref, e_ref, e2_ref, idx_ref):
    x = x_ref[...]
    ef = e_ref[...]
    xb = x.astype(jnp.bfloat16)
    eb = ef.astype(jnp.bfloat16)
    xe = lax.dot_general(xb, eb, (((1,), (0,)), ((), ())),
                         preferred_element_type=jnp.float32)
    x2 = x2_ref[0, 0, :][:, None]
    e2 = e2_ref[...]
    dist = jnp.sqrt(jnp.maximum((x2 - 2.0 * xe) + e2, 0.0))
    acc = None
    idx = None
    for w in range(_NE // _W):
        h = dist[:, w * _W:(w + 1) * _W]
        m = jnp.min(h, axis=1)
        i = jnp.argmin(h, axis=1).astype(jnp.int32) + w * _W
        mr = m.astype(jnp.bfloat16).astype(jnp.float32)
        if acc is None:
            acc, idx = mr, i
        else:
            upd = m < acc
            acc = jnp.where(upd, mr, acc)
            idx = jnp.where(upd, i, idx)
    idx_ref[0, 0, :] = idx


def _vq_indices(flat, x2, embed, e2):
    n = flat.shape[0]
    grid = n // _BM
    idx3 = pl.pallas_call(
        _dist_argmin_body,
        grid=(grid,),
        in_specs=[
            pl.BlockSpec((_BM, _DIM), lambda i: (i, 0)),
            pl.BlockSpec((1, 1, _BM), lambda i: (i, 0, 0)),
            pl.BlockSpec((_DIM, _NE), lambda i: (0, 0)),
            pl.BlockSpec((1, _NE), lambda i: (0, 0)),
        ],
        out_specs=pl.BlockSpec((1, 1, _BM), lambda i: (i, 0, 0)),
        out_shape=jax.ShapeDtypeStruct((grid, 1, _BM), jnp.int32),
    )(flat, x2.reshape(grid, 1, _BM), embed, e2.reshape(1, _NE))
    return idx3.reshape(n)


def _sc_gather(table, idx):
    b, d = idx.shape[0], table.shape[1]
    info = plsc.get_sparse_core_info()
    nc, ns = info.num_cores, info.num_subcores
    nw = nc * ns
    b_per_w = b // nw
    mesh = plsc.VectorSubcoreMesh(core_axis_name="c", subcore_axis_name="s")

    @functools.partial(
        pl.kernel, mesh=mesh,
        out_type=jax.ShapeDtypeStruct((b, d), table.dtype),
        scratch_types=[
            pltpu.VMEM((b_per_w,), jnp.int32),
            pltpu.VMEM((b_per_w, d), table.dtype),
            pltpu.SemaphoreType.DMA,
        ],
    )
    def k(table_hbm, idx_hbm, out_hbm, idx_v, rows_v, sem):
        wid = lax.axis_index("s") * nc + lax.axis_index("c")
        base = wid * b_per_w
        pltpu.sync_copy(idx_hbm.at[pl.ds(base, b_per_w)], idx_v)
        pltpu.async_copy(table_hbm.at[idx_v], rows_v, sem).wait()
        pltpu.sync_copy(rows_v, out_hbm.at[pl.ds(base, b_per_w)])

    return k(table, idx)


def kernel(input, embed):
    flat = input.reshape(-1, _DIM)
    x2 = jnp.sum(input * input, axis=-1).reshape(-1)
    e = embed.T
    e2 = jnp.sum(e * e, axis=1)
    table = jnp.pad(e, ((0, 0), (0, 128 - _DIM)))
    n = flat.shape[0]
    h = n // 2
    rows = []
    for lo, hi in ((0, h), (h, n)):
        idx = _vq_indices(flat[lo:hi], x2[lo:hi], embed, e2)
        rows.append(_sc_gather(table, idx))
    return jnp.concatenate(rows, axis=0)[:, :_DIM].reshape(input.shape)


# speedup vs baseline: 1.0433x; 1.0433x over previous
import functools

import jax
import jax.numpy as jnp
from jax import lax
from jax.experimental import pallas as pl
from jax.experimental.pallas import tpu as pltpu
from jax.experimental.pallas import tpu_sc as plsc

_DIM = 64
_NE = 8192
_W = 2048
_BM = 1024


def _dist_argmin_body(x_ref, x2_ref, e_ref, e2_ref, idx_ref):
    x = x_ref[...]
    ef = e_ref[...]
    xb = x.astype(jnp.bfloat16)
    eb = ef.astype(jnp.bfloat16)
    xe = lax.dot_general(xb, eb, (((1,), (0,)), ((), ())),
                         preferred_element_type=jnp.float32)
    x2 = x2_ref[0, 0, :][:, None]
    e2 = e2_ref[...]
    dist = jnp.sqrt(jnp.maximum((x2 - 2.0 * xe) + e2, 0.0))
    acc = None
    idx = None
    for w in range(_NE // _W):
        h = dist[:, w * _W:(w + 1) * _W]
        m = jnp.min(h, axis=1)
        i = jnp.argmin(h, axis=1).astype(jnp.int32) + w * _W
        mr = m.astype(jnp.bfloat16).astype(jnp.float32)
        if acc is None:
            acc, idx = mr, i
        else:
            upd = m < acc
            acc = jnp.where(upd, mr, acc)
            idx = jnp.where(upd, i, idx)
    idx_ref[0, 0, :] = idx


def _vq_indices(flat, x2, embed, e2):
    n = flat.shape[0]
    grid = n // _BM
    idx3 = pl.pallas_call(
        _dist_argmin_body,
        grid=(grid,),
        in_specs=[
            pl.BlockSpec((_BM, _DIM), lambda i: (i, 0)),
            pl.BlockSpec((1, 1, _BM), lambda i: (i, 0, 0)),
            pl.BlockSpec((_DIM, _NE), lambda i: (0, 0)),
            pl.BlockSpec((1, _NE), lambda i: (0, 0)),
        ],
        out_specs=pl.BlockSpec((1, 1, _BM), lambda i: (i, 0, 0)),
        out_shape=jax.ShapeDtypeStruct((grid, 1, _BM), jnp.int32),
    )(flat, x2.reshape(grid, 1, _BM), embed, e2.reshape(1, _NE))
    return idx3.reshape(n)


def _sc_gather(table, idx):
    b, d = idx.shape[0], table.shape[1]
    info = plsc.get_sparse_core_info()
    nc, ns = info.num_cores, info.num_subcores
    nw = nc * ns
    b_per_w = b // nw
    mesh = plsc.VectorSubcoreMesh(core_axis_name="c", subcore_axis_name="s")

    @functools.partial(
        pl.kernel, mesh=mesh,
        out_type=jax.ShapeDtypeStruct((b, d), table.dtype),
        scratch_types=[
            pltpu.VMEM((b_per_w,), jnp.int32),
            pltpu.VMEM((b_per_w, d), table.dtype),
            pltpu.SemaphoreType.DMA,
        ],
    )
    def k(table_hbm, idx_hbm, out_hbm, idx_v, rows_v, sem):
        wid = lax.axis_index("s") * nc + lax.axis_index("c")
        base = wid * b_per_w
        pltpu.sync_copy(idx_hbm.at[pl.ds(base, b_per_w)], idx_v)
        pltpu.async_copy(table_hbm.at[idx_v], rows_v, sem).wait()
        pltpu.sync_copy(rows_v, out_hbm.at[pl.ds(base, b_per_w)])

    return k(table, idx)


def kernel(input, embed):
    flat = input.reshape(-1, _DIM)
    x2 = jnp.sum(input * input, axis=-1).reshape(-1)
    e = embed.T
    e2 = jnp.sum(e * e, axis=1)
    table = jnp.pad(e, ((0, 0), (0, 128 - _DIM)))
    idx = _vq_indices(flat, x2, embed, e2)
    rows = _sc_gather(table, idx)
    return rows[:, :_DIM].reshape(input.shape)
